# Initial kernel scaffold; baseline (speedup 1.0000x reference)
import jax
import jax.numpy as jnp
from jax.experimental import pallas as pl


def kernel(pred, boxes, labels):
    raise NotImplementedError("write your pallas kernel here")


# speedup vs baseline: 3.8527x; 3.8527x over previous
import functools

import jax
import jax.numpy as jnp
from jax import lax
from jax.experimental import pallas as pl
from jax.experimental.pallas import tpu as pltpu
from jax.experimental.pallas import tpu_sc as plsc

B, C, H, W = 8, 85, 128, 128

_GATHER_DNUMS = lax.GatherDimensionNumbers(
    offset_dims=(), collapsed_slice_dims=(0,), start_index_map=(0,))


def _rot16(x, perm):
    return lax.gather(x, perm[:, None], _GATHER_DNUMS, slice_sizes=(1,),
                      mode=lax.GatherScatterMode.PROMISE_IN_BOUNDS)
NCLS = C - 5
NBOX = 50
NPAD = 64
NV = NPAD // 16
HW = H * W
CHW = C * HW


def _dense_body(x_ref, out_ref):
    x = x_ref[0]
    xb = jnp.clip(x[:4], -10.0, 10.0)
    xo = jnp.clip(x[4:5], -10.0, 10.0)
    xc = jnp.clip(x[5:], -10.0, 10.0)

    def softplus(v):
        return jnp.maximum(v, 0.0) + jnp.log1p(jnp.exp(-jnp.abs(v)))

    s_box = jnp.sum(xb * xb)
    s_obj = jnp.sum(softplus(xo))
    s_cls = jnp.sum(softplus(xc))
    lane = lax.broadcasted_iota(jnp.int32, (1, 128), 1)
    vals = jnp.where(lane == 0, s_box,
                     jnp.where(lane == 1, s_obj,
                               jnp.where(lane == 2, s_cls, 0.0)))
    out_ref[0] = vals


def _dense_sums(pred):
    return pl.pallas_call(
        _dense_body,
        grid=(B,),
        in_specs=[pl.BlockSpec((1, C, H, W), lambda i: (i, 0, 0, 0))],
        out_specs=pl.BlockSpec((1, 1, 128), lambda i: (i, 0, 0)),
        out_shape=jax.ShapeDtypeStruct((B, 1, 128), jnp.float32),
    )(pred)


def _sc_body(pred_hbm, bx_hbm, by_hbm, bw_hbm, bh_hbm, lab_hbm, out_hbm,
             bx_v, by_v, bw_v, bh_v, lab_v, cell_v, key2_v, tx_v, ty_v,
             w_v, w2_v, idx_v, val_v, out_v, sem):
    cid = lax.axis_index("c")
    sid = lax.axis_index("s")
    wid = sid * 2 + cid

    @pl.when(wid < B)
    def _():
        b = wid
        pltpu.sync_copy(bx_hbm.at[b], bx_v)
        pltpu.sync_copy(by_hbm.at[b], by_v)
        pltpu.sync_copy(bw_hbm.at[b], bw_v)
        pltpu.sync_copy(bh_hbm.at[b], bh_v)
        pltpu.sync_copy(lab_hbm.at[b], lab_v)

        lane16 = lax.iota(jnp.int32, 16)

        for v in range(NV):
            sl = pl.ds(v * 16, 16)
            gx = bx_v[sl] * float(W)
            gy = by_v[sl] * float(H)
            gi = jnp.minimum(gx.astype(jnp.int32), W - 1)
            gj = jnp.minimum(gy.astype(jnp.int32), H - 1)
            tx_v[sl] = gx - gi.astype(jnp.float32)
            ty_v[sl] = gy - gj.astype(jnp.float32)
            lane = lane16 + v * 16
            real = lane < NBOX
            cell = jnp.where(real, gj * W + gi, HW + 8 + lane)
            cell_v[sl] = cell
            key2_v[sl] = jnp.where(real, lab_v[sl] * HW + cell,
                                   NCLS * HW + 8 + lane)

        for a in range(NV):
            sla = pl.ds(a * 16, 16)
            ka = cell_v[sla]
            k2a = key2_v[sla]
            cnt = jnp.zeros((16,), jnp.float32)
            cnt2 = jnp.zeros((16,), jnp.float32)
            for bb in range(NV):
                kb = cell_v[pl.ds(bb * 16, 16)]
                k2b = key2_v[pl.ds(bb * 16, 16)]

                def rot_body(r, carry, kb=kb, k2b=k2b, ka=ka, k2a=k2a):
                    c1, c2 = carry
                    perm = jnp.bitwise_and(lane16 + r, 15)
                    kbr = _rot16(kb, perm)
                    k2br = _rot16(k2b, perm)
                    c1 = c1 + jnp.where(ka == kbr, 1.0, 0.0)
                    c2 = c2 + jnp.where(k2a == k2br, 1.0, 0.0)
                    return c1, c2

                cnt, cnt2 = lax.fori_loop(0, 16, rot_body, (cnt, cnt2))
            real = (lane16 + a * 16) < NBOX
            w_v[sla] = jnp.where(real, 1.0 / cnt, 0.0)
            w2_v[sla] = jnp.where(real, 1.0 / cnt2, 0.0)

        base = b * CHW
        zero16 = jnp.zeros((16,), jnp.float32)
        d_obj = zero16
        d_cls = zero16
        d_box = zero16

        t_refs = (tx_v, ty_v, bw_v, bh_v)

        for ch in range(4):
            for v in range(NV):
                sl = pl.ds(v * 16, 16)
                cg = jnp.minimum(cell_v[sl], HW - 1)
                idx_v[sl] = base + ch * HW + cg
            pltpu.async_copy(pred_hbm.at[idx_v], val_v, sem).wait()
            for v in range(NV):
                sl = pl.ds(v * 16, 16)
                x = jnp.clip(val_v[sl], -10.0, 10.0)
                t = t_refs[ch][sl]
                d_box = d_box + w_v[sl] * (t * t - 2.0 * x * t)

        for v in range(NV):
            sl = pl.ds(v * 16, 16)
            cg = jnp.minimum(cell_v[sl], HW - 1)
            idx_v[sl] = base + 4 * HW + cg
        pltpu.async_copy(pred_hbm.at[idx_v], val_v, sem).wait()
        for v in range(NV):
            sl = pl.ds(v * 16, 16)
            x = jnp.clip(val_v[sl], -10.0, 10.0)
            d_obj = d_obj - w_v[sl] * x

        for v in range(NV):
            sl = pl.ds(v * 16, 16)
            cg = jnp.minimum(cell_v[sl], HW - 1)
            idx_v[sl] = base + (5 + lab_v[sl]) * HW + cg
        pltpu.async_copy(pred_hbm.at[idx_v], val_v, sem).wait()
        for v in range(NV):
            sl = pl.ds(v * 16, 16)
            x = jnp.clip(val_v[sl], -10.0, 10.0)
            d_cls = d_cls - w2_v[sl] * x

        for shift in (8, 4, 2, 1):
            perm = jnp.bitwise_and(lane16 + shift, 15)
            d_obj = d_obj + _rot16(d_obj, perm)
            d_cls = d_cls + _rot16(d_cls, perm)
            d_box = d_box + _rot16(d_box, perm)

        out_v[...] = jnp.where(lane16 == 0, d_obj,
                               jnp.where(lane16 == 1, d_cls,
                                         jnp.where(lane16 == 2, d_box, 0.0)))
        pltpu.sync_copy(out_v, out_hbm.at[b])


@functools.partial(
    pl.kernel,
    out_type=jax.ShapeDtypeStruct((B, 16), jnp.float32),
    mesh=plsc.VectorSubcoreMesh(core_axis_name="c", subcore_axis_name="s",
                                num_cores=2, num_subcores=16),
    scratch_types=[
        pltpu.VMEM((NPAD,), jnp.float32),
        pltpu.VMEM((NPAD,), jnp.float32),
        pltpu.VMEM((NPAD,), jnp.float32),
        pltpu.VMEM((NPAD,), jnp.float32),
        pltpu.VMEM((NPAD,), jnp.int32),
        pltpu.VMEM((NPAD,), jnp.int32),
        pltpu.VMEM((NPAD,), jnp.int32),
        pltpu.VMEM((NPAD,), jnp.float32),
        pltpu.VMEM((NPAD,), jnp.float32),
        pltpu.VMEM((NPAD,), jnp.float32),
        pltpu.VMEM((NPAD,), jnp.float32),
        pltpu.VMEM((NPAD,), jnp.int32),
        pltpu.VMEM((NPAD,), jnp.float32),
        pltpu.VMEM((16,), jnp.float32),
        pltpu.SemaphoreType.DMA,
    ],
)
def _sc_corrections(pred_flat, bx, by, bw, bh, lab, out, *rest):
    _sc_body(pred_flat, bx, by, bw, bh, lab, out, *rest)


def kernel(pred, boxes, labels):
    pad = ((0, 0), (0, NPAD - NBOX))
    bx = jnp.pad(boxes[..., 0], pad)
    by = jnp.pad(boxes[..., 1], pad)
    bw = jnp.pad(boxes[..., 2], pad)
    bh = jnp.pad(boxes[..., 3], pad)
    lab = jnp.pad(labels.astype(jnp.int32), pad)

    tc = _dense_sums(pred)
    sc = _sc_corrections(pred.reshape(-1), bx, by, bw, bh, lab)

    s_box = jnp.sum(tc[:, 0, 0])
    s_obj = jnp.sum(tc[:, 0, 1])
    s_cls = jnp.sum(tc[:, 0, 2])
    d_obj = jnp.sum(sc[:, 0])
    d_cls = jnp.sum(sc[:, 1])
    d_box = jnp.sum(sc[:, 2])

    box_loss = (s_box + d_box) / (B * 4 * HW)
    obj_loss = (s_obj + d_obj) / (B * HW)
    cls_loss = (s_cls + d_cls) / (B * NCLS * HW)
    total = 5.0 * box_loss + obj_loss + cls_loss
    return jnp.clip(total, 0.0, 1000000.0)


# speedup vs baseline: 5.6085x; 1.4557x over previous
import functools

import jax
import jax.numpy as jnp
from jax import lax
from jax.experimental import pallas as pl
from jax.experimental.pallas import tpu as pltpu
from jax.experimental.pallas import tpu_sc as plsc

B, C, H, W = 8, 85, 128, 128

_GATHER_DNUMS = lax.GatherDimensionNumbers(
    offset_dims=(), collapsed_slice_dims=(0,), start_index_map=(0,))


def _rot16(x, perm):
    return lax.gather(x, perm[:, None], _GATHER_DNUMS, slice_sizes=(1,),
                      mode=lax.GatherScatterMode.PROMISE_IN_BOUNDS)
NCLS = C - 5
NBOX = 50
NPAD = 64
NV = NPAD // 16
HW = H * W
CHW = C * HW


_LOG2E = 1.4426950408889634
_LN2 = 0.6931471805599453
_OBJ_W = 1.0 / (8 * 128 * 128)
_CLS_W = 1.0 / (8 * 80 * 128 * 128)


def _dense_body(x_ref, out_ref):
    x = x_ref[0]
    xb = jnp.clip(x[:4], -10.0, 10.0)
    s_box = jnp.sum(xb * xb)
    xoc = jnp.clip(x[4:], -10.0, 10.0)
    sp = jnp.log(1.0 + jnp.exp2(xoc * _LOG2E))
    s_spw = _CLS_W * jnp.sum(sp) + (_OBJ_W - _CLS_W) * jnp.sum(sp[0])
    lane = lax.broadcasted_iota(jnp.int32, (1, 128), 1)
    vals = jnp.where(lane == 0, s_box,
                     jnp.where(lane == 1, s_spw, 0.0))
    out_ref[0] = vals


def _dense_sums(pred):
    return pl.pallas_call(
        _dense_body,
        grid=(B,),
        in_specs=[pl.BlockSpec((1, C, H, W), lambda i: (i, 0, 0, 0))],
        out_specs=pl.BlockSpec((1, 1, 128), lambda i: (i, 0, 0)),
        out_shape=jax.ShapeDtypeStruct((B, 1, 128), jnp.float32),
    )(pred)


def _sc_body(pred_hbm, bx_hbm, by_hbm, bw_hbm, bh_hbm, lab_hbm, out_hbm,
             bx_v, by_v, bw_v, bh_v, lab_v, cell_v, key2_v, tx_v, ty_v,
             w_v, w2_v, idx_v, val_v, out_v, sem):
    cid = lax.axis_index("c")
    sid = lax.axis_index("s")
    wid = sid * 2 + cid

    @pl.when(wid < B)
    def _():
        b = wid
        pltpu.sync_copy(bx_hbm.at[b], bx_v)
        pltpu.sync_copy(by_hbm.at[b], by_v)
        pltpu.sync_copy(bw_hbm.at[b], bw_v)
        pltpu.sync_copy(bh_hbm.at[b], bh_v)
        pltpu.sync_copy(lab_hbm.at[b], lab_v)

        lane16 = lax.iota(jnp.int32, 16)

        for v in range(NV):
            sl = pl.ds(v * 16, 16)
            gx = bx_v[sl] * float(W)
            gy = by_v[sl] * float(H)
            gi = jnp.minimum(gx.astype(jnp.int32), W - 1)
            gj = jnp.minimum(gy.astype(jnp.int32), H - 1)
            tx_v[sl] = gx - gi.astype(jnp.float32)
            ty_v[sl] = gy - gj.astype(jnp.float32)
            lane = lane16 + v * 16
            real = lane < NBOX
            cell = jnp.where(real, gj * W + gi, HW + 8 + lane)
            cell_v[sl] = cell
            key2_v[sl] = jnp.where(real, lab_v[sl] * HW + cell,
                                   NCLS * HW + 8 + lane)

        for a in range(NV):
            sla = pl.ds(a * 16, 16)
            ka = cell_v[sla]
            k2a = key2_v[sla]
            cnt = jnp.zeros((16,), jnp.float32)
            cnt2 = jnp.zeros((16,), jnp.float32)
            for bb in range(NV):
                kb = cell_v[pl.ds(bb * 16, 16)]
                k2b = key2_v[pl.ds(bb * 16, 16)]

                def rot_body(r, carry, kb=kb, k2b=k2b, ka=ka, k2a=k2a):
                    c1, c2 = carry
                    perm = jnp.bitwise_and(lane16 + r, 15)
                    kbr = _rot16(kb, perm)
                    k2br = _rot16(k2b, perm)
                    c1 = c1 + jnp.where(ka == kbr, 1.0, 0.0)
                    c2 = c2 + jnp.where(k2a == k2br, 1.0, 0.0)
                    return c1, c2

                cnt, cnt2 = lax.fori_loop(0, 16, rot_body, (cnt, cnt2))
            real = (lane16 + a * 16) < NBOX
            w_v[sla] = jnp.where(real, 1.0 / cnt, 0.0)
            w2_v[sla] = jnp.where(real, 1.0 / cnt2, 0.0)

        base = b * CHW
        zero16 = jnp.zeros((16,), jnp.float32)
        d_obj = zero16
        d_cls = zero16
        d_box = zero16

        t_refs = (tx_v, ty_v, bw_v, bh_v)

        for ch in range(4):
            for v in range(NV):
                sl = pl.ds(v * 16, 16)
                cg = jnp.minimum(cell_v[sl], HW - 1)
                idx_v[sl] = base + ch * HW + cg
            pltpu.async_copy(pred_hbm.at[idx_v], val_v, sem).wait()
            for v in range(NV):
                sl = pl.ds(v * 16, 16)
                x = jnp.clip(val_v[sl], -10.0, 10.0)
                t = t_refs[ch][sl]
                d_box = d_box + w_v[sl] * (t * t - 2.0 * x * t)

        for v in range(NV):
            sl = pl.ds(v * 16, 16)
            cg = jnp.minimum(cell_v[sl], HW - 1)
            idx_v[sl] = base + 4 * HW + cg
        pltpu.async_copy(pred_hbm.at[idx_v], val_v, sem).wait()
        for v in range(NV):
            sl = pl.ds(v * 16, 16)
            x = jnp.clip(val_v[sl], -10.0, 10.0)
            d_obj = d_obj - w_v[sl] * x

        for v in range(NV):
            sl = pl.ds(v * 16, 16)
            cg = jnp.minimum(cell_v[sl], HW - 1)
            idx_v[sl] = base + (5 + lab_v[sl]) * HW + cg
        pltpu.async_copy(pred_hbm.at[idx_v], val_v, sem).wait()
        for v in range(NV):
            sl = pl.ds(v * 16, 16)
            x = jnp.clip(val_v[sl], -10.0, 10.0)
            d_cls = d_cls - w2_v[sl] * x

        for shift in (8, 4, 2, 1):
            perm = jnp.bitwise_and(lane16 + shift, 15)
            d_obj = d_obj + _rot16(d_obj, perm)
            d_cls = d_cls + _rot16(d_cls, perm)
            d_box = d_box + _rot16(d_box, perm)

        out_v[...] = jnp.where(lane16 == 0, d_obj,
                               jnp.where(lane16 == 1, d_cls,
                                         jnp.where(lane16 == 2, d_box, 0.0)))
        pltpu.sync_copy(out_v, out_hbm.at[b])


@functools.partial(
    pl.kernel,
    out_type=jax.ShapeDtypeStruct((B, 16), jnp.float32),
    mesh=plsc.VectorSubcoreMesh(core_axis_name="c", subcore_axis_name="s",
                                num_cores=2, num_subcores=16),
    scratch_types=[
        pltpu.VMEM((NPAD,), jnp.float32),
        pltpu.VMEM((NPAD,), jnp.float32),
        pltpu.VMEM((NPAD,), jnp.float32),
        pltpu.VMEM((NPAD,), jnp.float32),
        pltpu.VMEM((NPAD,), jnp.int32),
        pltpu.VMEM((NPAD,), jnp.int32),
        pltpu.VMEM((NPAD,), jnp.int32),
        pltpu.VMEM((NPAD,), jnp.float32),
        pltpu.VMEM((NPAD,), jnp.float32),
        pltpu.VMEM((NPAD,), jnp.float32),
        pltpu.VMEM((NPAD,), jnp.float32),
        pltpu.VMEM((NPAD,), jnp.int32),
        pltpu.VMEM((NPAD,), jnp.float32),
        pltpu.VMEM((16,), jnp.float32),
        pltpu.SemaphoreType.DMA,
    ],
)
def _sc_corrections(pred_flat, bx, by, bw, bh, lab, out, *rest):
    _sc_body(pred_flat, bx, by, bw, bh, lab, out, *rest)


def kernel(pred, boxes, labels):
    pad = ((0, 0), (0, NPAD - NBOX))
    bx = jnp.pad(boxes[..., 0], pad)
    by = jnp.pad(boxes[..., 1], pad)
    bw = jnp.pad(boxes[..., 2], pad)
    bh = jnp.pad(boxes[..., 3], pad)
    lab = jnp.pad(labels.astype(jnp.int32), pad)

    tc = _dense_sums(pred)
    sc = _sc_corrections(pred.reshape(-1), bx, by, bw, bh, lab)

    s_box = jnp.sum(tc[:, 0, 0])
    s_spw = jnp.sum(tc[:, 0, 1])
    d_obj = jnp.sum(sc[:, 0])
    d_cls = jnp.sum(sc[:, 1])
    d_box = jnp.sum(sc[:, 2])

    total = (5.0 * (s_box + d_box) / (B * 4 * HW)
             + s_spw + d_obj * _OBJ_W + d_cls * _CLS_W)
    return jnp.clip(total, 0.0, 1000000.0)
